# baseline (speedup 1.0000x reference)
import jax
import jax.numpy as jnp
from jax import lax
from jax.experimental import pallas as pl
from jax.experimental.pallas import tpu as pltpu

_T = 0.05
_C = 3072
_NU = 32
_TAIL = ((98304, 0, 512), (98816, 512, 512), (99328, 1024, 512), (99840, 1536, 160))
_TROWS = 1696


def _dot(x, m):
    return jax.lax.dot_general(
        x, m, dimension_numbers=(((1,), (1,)), ((), ())),
        preferred_element_type=jnp.float32) / _T


def _body(x_ref, m_hbm, o_hbm, buf0, buf1, tbuf, ob0, ob1, otb,
          si0, si1, sit, so0, so1, sot):
    x = x_ref[...]
    n_pairs = _NU // 2

    def in_copy(buf, sem, off):
        return pltpu.make_async_copy(m_hbm.at[pl.ds(off, _C)], buf, sem)

    def out_copy(ob, sem, off):
        return pltpu.make_async_copy(ob, o_hbm.at[:, pl.ds(off, _C)], sem)

    in_copy(buf0, si0, 0).start()
    in_copy(buf1, si1, _C).start()

    def loop(j, carry):
        off0 = 2 * j * _C
        off1 = off0 + _C

        in_copy(buf0, si0, off0).wait()

        @pl.when(j > 0)
        def _():
            out_copy(ob0, so0, off0 - 2 * _C).wait()

        ob0[...] = _dot(x, buf0[...])

        @pl.when(j < n_pairs - 1)
        def _():
            in_copy(buf0, si0, off0 + 2 * _C).start()

        @pl.when(j == n_pairs - 1)
        def _():
            for hbm_off, loc_off, cn in _TAIL:
                pltpu.make_async_copy(
                    m_hbm.at[pl.ds(hbm_off, cn)],
                    tbuf.at[pl.ds(loc_off, cn)], sit).start()

        out_copy(ob0, so0, off0).start()

        in_copy(buf1, si1, off1).wait()

        @pl.when(j > 0)
        def _():
            out_copy(ob1, so1, off1 - 2 * _C).wait()

        ob1[...] = _dot(x, buf1[...])

        @pl.when(j < n_pairs - 1)
        def _():
            in_copy(buf1, si1, off1 + 2 * _C).start()

        out_copy(ob1, so1, off1).start()
        return carry

    lax.fori_loop(0, n_pairs, loop, 0)

    for t, (hbm_off, loc_off, cn) in enumerate(_TAIL):
        pltpu.make_async_copy(
            m_hbm.at[pl.ds(hbm_off, cn)], tbuf.at[pl.ds(loc_off, cn)], sit).wait()
        otb[:, pl.ds(loc_off, cn)] = _dot(x, tbuf[pl.ds(loc_off, cn), :])
    pltpu.make_async_copy(
        otb, o_hbm.at[:, pl.ds(98304, _TROWS)], sot).start()

    out_copy(ob0, so0, (_NU - 2) * _C).wait()
    out_copy(ob1, so1, (_NU - 1) * _C).wait()
    pltpu.make_async_copy(otb, o_hbm.at[:, pl.ds(98304, _TROWS)], sot).wait()


def kernel(x, memory):
    b, k = x.shape
    n = memory.shape[0]
    return pl.pallas_call(
        _body,
        in_specs=[
            pl.BlockSpec((b, k), lambda: (0, 0)),
            pl.BlockSpec(memory_space=pl.ANY),
        ],
        out_specs=pl.BlockSpec(memory_space=pl.ANY),
        out_shape=jax.ShapeDtypeStruct((b, n), jnp.float32),
        scratch_shapes=[
            pltpu.VMEM((_C, k), jnp.float32),
            pltpu.VMEM((_C, k), jnp.float32),
            pltpu.VMEM((_TROWS, k), jnp.float32),
            pltpu.VMEM((b, _C), jnp.float32),
            pltpu.VMEM((b, _C), jnp.float32),
            pltpu.VMEM((b, _TROWS), jnp.float32),
            pltpu.SemaphoreType.DMA,
            pltpu.SemaphoreType.DMA,
            pltpu.SemaphoreType.DMA,
            pltpu.SemaphoreType.DMA,
            pltpu.SemaphoreType.DMA,
            pltpu.SemaphoreType.DMA,
        ],
        compiler_params=pltpu.CompilerParams(
            vmem_limit_bytes=100 * 1024 * 1024),
    )(x, memory)


# speedup vs baseline: 1.0115x; 1.0115x over previous
import jax
import jax.numpy as jnp
from jax import lax
from jax.experimental import pallas as pl
from jax.experimental.pallas import tpu as pltpu

_T = 0.05
_C = 2048
_NU = 48
_TAIL = ((98304, 0, 512), (98816, 512, 512), (99328, 1024, 512), (99840, 1536, 160))
_TROWS = 1696


def _dot(x, m):
    return jax.lax.dot_general(
        x, m, dimension_numbers=(((1,), (1,)), ((), ())),
        preferred_element_type=jnp.float32) / _T


def _body(x_ref, m_hbm, o_hbm, buf0, buf1, tbuf, ob0, ob1, otb,
          si0, si1, sit, so0, so1, sot):
    x = x_ref[...]
    n_pairs = _NU // 2

    def in_copy(buf, sem, off):
        return pltpu.make_async_copy(m_hbm.at[pl.ds(off, _C)], buf, sem)

    def out_copy(ob, sem, off):
        return pltpu.make_async_copy(ob, o_hbm.at[:, pl.ds(off, _C)], sem)

    in_copy(buf0, si0, 0).start()
    in_copy(buf1, si1, _C).start()

    def loop(j, carry):
        off0 = 2 * j * _C
        off1 = off0 + _C

        in_copy(buf0, si0, off0).wait()

        @pl.when(j > 0)
        def _():
            out_copy(ob0, so0, off0 - 2 * _C).wait()

        ob0[...] = _dot(x, buf0[...])

        @pl.when(j < n_pairs - 1)
        def _():
            in_copy(buf0, si0, off0 + 2 * _C).start()

        @pl.when(j == n_pairs - 1)
        def _():
            for hbm_off, loc_off, cn in _TAIL:
                pltpu.make_async_copy(
                    m_hbm.at[pl.ds(hbm_off, cn)],
                    tbuf.at[pl.ds(loc_off, cn)], sit).start()

        out_copy(ob0, so0, off0).start()

        in_copy(buf1, si1, off1).wait()

        @pl.when(j > 0)
        def _():
            out_copy(ob1, so1, off1 - 2 * _C).wait()

        ob1[...] = _dot(x, buf1[...])

        @pl.when(j < n_pairs - 1)
        def _():
            in_copy(buf1, si1, off1 + 2 * _C).start()

        out_copy(ob1, so1, off1).start()
        return carry

    lax.fori_loop(0, n_pairs, loop, 0)

    for t, (hbm_off, loc_off, cn) in enumerate(_TAIL):
        pltpu.make_async_copy(
            m_hbm.at[pl.ds(hbm_off, cn)], tbuf.at[pl.ds(loc_off, cn)], sit).wait()
        otb[:, pl.ds(loc_off, cn)] = _dot(x, tbuf[pl.ds(loc_off, cn), :])
    pltpu.make_async_copy(
        otb, o_hbm.at[:, pl.ds(98304, _TROWS)], sot).start()

    out_copy(ob0, so0, (_NU - 2) * _C).wait()
    out_copy(ob1, so1, (_NU - 1) * _C).wait()
    pltpu.make_async_copy(otb, o_hbm.at[:, pl.ds(98304, _TROWS)], sot).wait()


def kernel(x, memory):
    b, k = x.shape
    n = memory.shape[0]
    return pl.pallas_call(
        _body,
        in_specs=[
            pl.BlockSpec((b, k), lambda: (0, 0)),
            pl.BlockSpec(memory_space=pl.ANY),
        ],
        out_specs=pl.BlockSpec(memory_space=pl.ANY),
        out_shape=jax.ShapeDtypeStruct((b, n), jnp.float32),
        scratch_shapes=[
            pltpu.VMEM((_C, k), jnp.float32),
            pltpu.VMEM((_C, k), jnp.float32),
            pltpu.VMEM((_TROWS, k), jnp.float32),
            pltpu.VMEM((b, _C), jnp.float32),
            pltpu.VMEM((b, _C), jnp.float32),
            pltpu.VMEM((b, _TROWS), jnp.float32),
            pltpu.SemaphoreType.DMA,
            pltpu.SemaphoreType.DMA,
            pltpu.SemaphoreType.DMA,
            pltpu.SemaphoreType.DMA,
            pltpu.SemaphoreType.DMA,
            pltpu.SemaphoreType.DMA,
        ],
        compiler_params=pltpu.CompilerParams(
            vmem_limit_bytes=100 * 1024 * 1024),
    )(x, memory)
